# Initial kernel scaffold; baseline (speedup 1.0000x reference)
import jax
import jax.numpy as jnp
from jax.experimental import pallas as pl


def kernel(V, E, edges, W_f, W_a, b_a):
    raise NotImplementedError("write your pallas kernel here")


# speedup vs baseline: 2601.9055x; 2601.9055x over previous
import functools
import jax
import jax.numpy as jnp
from jax import lax
from jax.experimental import pallas as pl
from jax.experimental.pallas import tpu as pltpu
from jax.experimental.pallas import tpu_sc as plsc

NC, NS, L = 2, 16, 16
NW = NC * NS
NEG = -1.0e30


def _node_body(v_ref, wt_ref, w12_ref, hv_ref, asr_ref):
    hv = jnp.dot(v_ref[...], wt_ref[...], preferred_element_type=jnp.float32)
    hv_ref[...] = hv
    asr_ref[...] = jnp.dot(hv, w12_ref[...], preferred_element_type=jnp.float32)


def _edge_body(e_ref, w3_ref, b_ref, ae_ref):
    ae_ref[...] = (
        jnp.dot(e_ref[...], w3_ref[...], preferred_element_type=jnp.float32)
        + b_ref[0, 0]
    )


def _combine_body(hv_ref, bins_ref, mx_ref, out_ref):
    mx = mx_ref[...]
    m_all = jnp.max(mx)
    scale = jnp.exp(mx[:, 0:1] - m_all)
    denom = lax.dot_general(
        bins_ref[...], scale,
        dimension_numbers=(((0,), (0,)), ((), ())),
        preferred_element_type=jnp.float32,
    )
    out_ref[...] = hv_ref[...] * (denom / (denom + 1e-8))


def _make_sc_kernel(n_pad, epw):
    mesh = plsc.VectorSubcoreMesh(core_axis_name="c", subcore_axis_name="s")

    @functools.partial(
        pl.kernel,
        mesh=mesh,
        compiler_params=pltpu.CompilerParams(needs_layout_passes=False),
        out_type=(
            jax.ShapeDtypeStruct((NW, n_pad), jnp.float32),
            jax.ShapeDtypeStruct((NW, L), jnp.float32),
        ),
        scratch_types=[
            pltpu.VMEM((n_pad,), jnp.float32),
            pltpu.VMEM((n_pad,), jnp.float32),
            pltpu.VMEM((epw,), jnp.int32),
            pltpu.VMEM((epw,), jnp.int32),
            pltpu.VMEM((epw,), jnp.float32),
            pltpu.VMEM((epw,), jnp.float32),
            pltpu.VMEM((n_pad,), jnp.float32),
            pltpu.VMEM((L,), jnp.float32),
        ],
    )
    def sc_kernel(as_hbm, ar_hbm, src_hbm, dst_hbm, ae_hbm,
                  bins_out, mx_out,
                  as_v, ar_v, src_v, dst_v, ae_v, logit_v, bins_v, mx_v):
        wid = lax.axis_index("s") * NC + lax.axis_index("c")
        base = wid * epw
        pltpu.sync_copy(as_hbm, as_v)
        pltpu.sync_copy(ar_hbm, ar_v)
        pltpu.sync_copy(src_hbm.at[pl.ds(base, epw)], src_v)
        pltpu.sync_copy(dst_hbm.at[pl.ds(base, epw)], dst_v)
        pltpu.sync_copy(ae_hbm.at[pl.ds(base, epw)], ae_v)

        def zero_body(i, carry):
            bins_v[pl.ds(i * L, L)] = jnp.zeros((L,), jnp.float32)
            return carry
        lax.fori_loop(0, n_pad // L, zero_body, 0)

        def logit_body(i, m):
            sl = pl.ds(i * L, L)
            a = plsc.load_gather(as_v, [src_v[sl]])
            r = plsc.load_gather(ar_v, [dst_v[sl]])
            lg = a + r + ae_v[sl]
            lg = jnp.maximum(lg, lg * 0.2)
            logit_v[sl] = lg
            return jnp.maximum(m, lg)
        m = lax.fori_loop(0, epw // L, logit_body,
                          jnp.full((L,), NEG, jnp.float32))
        m_loc = jnp.max(m)
        mx_v[...] = jnp.zeros((L,), jnp.float32) + m_loc

        def accum_body(i, carry):
            sl = pl.ds(i * L, L)
            att = jnp.exp(logit_v[sl] - m_loc)
            plsc.addupdate_scatter(bins_v, [src_v[sl]], att)
            return carry
        lax.fori_loop(0, epw // L, accum_body, 0)

        pltpu.sync_copy(bins_v, bins_out.at[wid])
        pltpu.sync_copy(mx_v, mx_out.at[wid])

    return sc_kernel


def kernel(V, E, edges, W_f, W_a, b_a):
    B, n_nodes, d_feat = V.shape
    n_edges = edges.shape[1]
    d_out = W_f.shape[0]

    blk = 1024
    n_pad = ((n_nodes + blk - 1) // blk) * blk
    epw = ((n_edges // NW + L - 1) // L) * L
    e_pad = epw * NW

    v2 = V[0]
    e2 = E[0]
    w1 = W_a[0, :d_out]
    w2 = W_a[0, d_out:2 * d_out]
    w3 = W_a[0, 2 * d_out:]

    vp = jnp.zeros((n_pad, d_feat), jnp.float32).at[:n_nodes].set(v2)
    w12 = jnp.stack([w1, w2], axis=1)

    hv, asr = pl.pallas_call(
        _node_body,
        grid=(n_pad // blk,),
        in_specs=[
            pl.BlockSpec((blk, d_feat), lambda i: (i, 0)),
            pl.BlockSpec((d_feat, d_out), lambda i: (0, 0)),
            pl.BlockSpec((d_out, 2), lambda i: (0, 0)),
        ],
        out_specs=[
            pl.BlockSpec((blk, d_out), lambda i: (i, 0)),
            pl.BlockSpec((blk, 2), lambda i: (i, 0)),
        ],
        out_shape=[
            jax.ShapeDtypeStruct((n_pad, d_out), jnp.float32),
            jax.ShapeDtypeStruct((n_pad, 2), jnp.float32),
        ],
    )(vp, W_f.T, w12)

    eblk = 6400
    ae = pl.pallas_call(
        _edge_body,
        grid=(n_edges // eblk,),
        in_specs=[
            pl.BlockSpec((eblk, d_feat), lambda i: (i, 0)),
            pl.BlockSpec((d_feat, 1), lambda i: (0, 0)),
            pl.BlockSpec((1, 1), lambda i: (0, 0)),
        ],
        out_specs=pl.BlockSpec((eblk, 1), lambda i: (i, 0)),
        out_shape=jax.ShapeDtypeStruct((n_edges, 1), jnp.float32),
    )(e2, w3[:, None], b_a[:, None])

    a_s = asr[:, 0]
    a_r = asr[:, 1]
    src = jnp.concatenate(
        [edges[0, :, 0], jnp.zeros((e_pad - n_edges,), jnp.int32)])
    dst = jnp.concatenate(
        [edges[0, :, 1], jnp.zeros((e_pad - n_edges,), jnp.int32)])
    ae_p = jnp.concatenate(
        [ae[:, 0], jnp.full((e_pad - n_edges,), NEG, jnp.float32)])

    bins, mx = _make_sc_kernel(n_pad, epw)(a_s, a_r, src, dst, ae_p)

    h_full = pl.pallas_call(
        _combine_body,
        grid=(n_pad // blk,),
        in_specs=[
            pl.BlockSpec((blk, d_out), lambda i: (i, 0)),
            pl.BlockSpec((NW, blk), lambda i: (0, i)),
            pl.BlockSpec((NW, L), lambda i: (0, 0)),
        ],
        out_specs=pl.BlockSpec((blk, d_out), lambda i: (i, 0)),
        out_shape=jax.ShapeDtypeStruct((n_pad, d_out), jnp.float32),
    )(hv, bins, mx)

    return h_full[:n_nodes][None]
